# Initial kernel scaffold; baseline (speedup 1.0000x reference)
import jax
import jax.numpy as jnp
from jax.experimental import pallas as pl


def kernel(hidden_states, gate_w, shared_gate_w, Wgp, Wup, Wdn, Sgp, Sup, Sdn):
    raise NotImplementedError("write your pallas kernel here")


# speedup vs baseline: 1.1727x; 1.1727x over previous
import functools
import math

import jax
import jax.numpy as jnp
from jax.experimental import pallas as pl

T = 2048
D = 1024
E = 8
DFF = 1024
TOP_K = 2
_SCALE = 1.0 / math.sqrt(TOP_K)


def _router_body(x_ref, gate_ref, sgw_ref, w_ref, gs_ref):
    x = x_ref[...].astype(jnp.bfloat16)
    logits = jax.lax.dot_general(
        x, gate_ref[...].astype(jnp.bfloat16), (((1,), (1,)), ((), ())),
        preferred_element_type=jnp.float32)
    p = jax.nn.softmax(logits, axis=-1)
    m1 = jnp.max(p, axis=-1, keepdims=True)
    p_rest = jnp.where(p >= m1, -jnp.inf, p)
    m2 = jnp.max(p_rest, axis=-1, keepdims=True)
    mask = p >= m2
    pm = jnp.where(mask, p, 0.0)
    w_ref[...] = pm / jnp.sum(pm, axis=-1, keepdims=True)
    gs = jnp.sum(x_ref[...] * sgw_ref[...], axis=1, keepdims=True)
    gs_ref[...] = jax.nn.sigmoid(gs)


BT = 512
NT = T // BT


def _moe_body(xb_ref, w_ref, gs_ref, wgp_ref, wup_ref, wdn_ref,
              sgp_ref, sup_ref, sdn_ref, out_ref):
    e = pl.program_id(0)
    t = pl.program_id(1)
    rows = pl.ds(t * BT, BT)

    def mlp(x, wg, wu, wd):
        g = jax.lax.dot_general(x, wg, (((1,), (1,)), ((), ())),
                                preferred_element_type=jnp.float32)
        u = jax.lax.dot_general(x, wu, (((1,), (1,)), ((), ())),
                                preferred_element_type=jnp.float32)
        h = (g * jax.nn.sigmoid(g) * u).astype(jnp.bfloat16)
        return jax.lax.dot_general(h, wd, (((1,), (1,)), ((), ())),
                                   preferred_element_type=jnp.float32)

    xb = xb_ref[...]

    @pl.when(e == 0)
    def _init():
        shared = mlp(xb, sgp_ref[...], sup_ref[...], sdn_ref[...])
        out_ref[rows, :] = gs_ref[...] * shared

    o = mlp(xb, wgp_ref[0], wup_ref[0], wdn_ref[0])
    lane = jax.lax.broadcasted_iota(jnp.int32, (1, E), 1)
    we = jnp.sum(jnp.where(lane == e, w_ref[...], 0.0), axis=1, keepdims=True)
    acc = out_ref[rows, :] + we * o

    @pl.when(e == E - 1)
    def _fin():
        out_ref[rows, :] = acc * _SCALE

    @pl.when(e < E - 1)
    def _acc():
        out_ref[rows, :] = acc


@jax.jit
def kernel(hidden_states, gate_w, shared_gate_w, Wgp, Wup, Wdn, Sgp, Sup, Sdn):
    x = hidden_states.reshape(T, D)

    wdense, gscore = pl.pallas_call(
        _router_body,
        out_shape=(jax.ShapeDtypeStruct((T, E), jnp.float32),
                   jax.ShapeDtypeStruct((T, 1), jnp.float32)),
    )(x, gate_w, shared_gate_w)

    xb = x.astype(jnp.bfloat16)
    Wgp16 = Wgp.astype(jnp.bfloat16)
    Wup16 = Wup.astype(jnp.bfloat16)
    Wdn16 = Wdn.astype(jnp.bfloat16)
    Sgp16 = Sgp.astype(jnp.bfloat16)
    Sup16 = Sup.astype(jnp.bfloat16)
    Sdn16 = Sdn.astype(jnp.bfloat16)

    expert_spec = lambda d0, d1: pl.BlockSpec((1, d0, d1),
                                              lambda e, t: (e, 0, 0))
    const_spec = lambda shape: pl.BlockSpec(shape,
                                            lambda e, t: (0,) * len(shape))
    tok_spec = lambda d1: pl.BlockSpec((BT, d1), lambda e, t: (t, 0))

    out = pl.pallas_call(
        _moe_body,
        grid=(E, NT),
        in_specs=[
            tok_spec(D),
            tok_spec(E),
            tok_spec(1),
            expert_spec(DFF, D),
            expert_spec(DFF, D),
            expert_spec(D, DFF),
            const_spec((DFF, D)),
            const_spec((DFF, D)),
            const_spec((D, DFF)),
        ],
        out_specs=pl.BlockSpec((T, D), lambda e, t: (0, 0)),
        out_shape=jax.ShapeDtypeStruct((T, D), jnp.float32),
    )(xb, wdense, gscore, Wgp16, Wup16, Wdn16, Sgp16, Sup16, Sdn16)
    return out


# speedup vs baseline: 1.5006x; 1.2796x over previous
import math

import jax
import jax.numpy as jnp
from jax.experimental import pallas as pl

T = 2048
D = 1024
E = 8
DFF = 1024
TOP_K = 2
_SCALE = 1.0 / math.sqrt(TOP_K)

BT = 512
NT = T // BT


def _dot_t(a, b):
    return jax.lax.dot_general(a, b, (((1,), (1,)), ((), ())),
                               preferred_element_type=jnp.float32)


def _router_body(x_ref, gate_ref, sgw_ref, sgp_ref, sup_ref, sdn_ref,
                 w_ref, gsh_ref):
    x = x_ref[...]
    logits = _dot_t(x, gate_ref[...])
    p = jax.nn.softmax(logits, axis=-1)
    m1 = jnp.max(p, axis=-1, keepdims=True)
    p_rest = jnp.where(p >= m1, -jnp.inf, p)
    m2 = jnp.max(p_rest, axis=-1, keepdims=True)
    mask = p >= m2
    pm = jnp.where(mask, p, 0.0)
    w_ref[...] = pm / jnp.sum(pm, axis=-1, keepdims=True)
    gs = jax.nn.sigmoid(jnp.sum(x * sgw_ref[...], axis=1, keepdims=True))
    g = _dot_t(x, sgp_ref[...])
    u = _dot_t(x, sup_ref[...])
    h = g * jax.nn.sigmoid(g) * u
    gsh_ref[...] = gs * _dot_t(h, sdn_ref[...])


def _moe_body(x_ref, w_ref, gsh_ref, wgp_ref, wup_ref, wdn_ref, out_ref):
    e = pl.program_id(0)
    t = pl.program_id(1)
    rows = pl.ds(t * BT, BT)

    x = x_ref[...]
    g = _dot_t(x, wgp_ref[0])
    u = _dot_t(x, wup_ref[0])
    h = g * jax.nn.sigmoid(g) * u
    o = _dot_t(h, wdn_ref[0])

    lane = jax.lax.broadcasted_iota(jnp.int32, (1, E), 1)
    we = jnp.sum(jnp.where(lane == e, w_ref[...], 0.0), axis=1, keepdims=True)

    @pl.when(e == 0)
    def _init():
        out_ref[rows, :] = gsh_ref[...] + we * o

    @pl.when(jnp.logical_and(e > 0, e < E - 1))
    def _acc():
        out_ref[rows, :] += we * o

    @pl.when(e == E - 1)
    def _fin():
        out_ref[rows, :] = (out_ref[rows, :] + we * o) * _SCALE


@jax.jit
def kernel(hidden_states, gate_w, shared_gate_w, Wgp, Wup, Wdn, Sgp, Sup, Sdn):
    x = hidden_states.reshape(T, D)

    tok = lambda d1: pl.BlockSpec((BT, d1), lambda t: (t, 0))
    full = lambda s: pl.BlockSpec(s, lambda t: (0,) * len(s))

    wdense, gshared = pl.pallas_call(
        _router_body,
        grid=(NT,),
        in_specs=[tok(D), full((E, D)), full((1, D)),
                  full((DFF, D)), full((DFF, D)), full((D, DFF))],
        out_specs=(tok(E), tok(D)),
        out_shape=(jax.ShapeDtypeStruct((T, E), jnp.float32),
                   jax.ShapeDtypeStruct((T, D), jnp.float32)),
    )(x, gate_w, shared_gate_w, Sgp, Sup, Sdn)

    tok2 = lambda d1: pl.BlockSpec((BT, d1), lambda e, t: (t, 0))
    exp2 = lambda d0, d1: pl.BlockSpec((1, d0, d1), lambda e, t: (e, 0, 0))

    out = pl.pallas_call(
        _moe_body,
        grid=(E, NT),
        in_specs=[tok2(D), tok2(E), tok2(D),
                  exp2(DFF, D), exp2(DFF, D), exp2(D, DFF)],
        out_specs=pl.BlockSpec((T, D), lambda e, t: (0, 0)),
        out_shape=jax.ShapeDtypeStruct((T, D), jnp.float32),
    )(x, wdense, gshared, Wgp, Wup, Wdn)
    return out
